# baseline (speedup 1.0000x reference)
import functools

import jax
import jax.numpy as jnp
from jax import lax
from jax.experimental import pallas as pl
from jax.experimental.pallas import tpu as pltpu
from jax.experimental.pallas import tpu_sc as plsc

NC = 2
NS = 16
NW = NC * NS

D = 64
NB = 4096
S = 50
BW = NB // NW
NBUF = 2
L = 16


def _emb_body(
    idx_hbm, tab_hbm, out_hbm, bidx_v, pidx_v, half_v, rows_v, st_v, gsem, wsem
):
    wid = lax.axis_index("s") * NC + lax.axis_index("c")
    b0 = wid * BW
    pltpu.sync_copy(idx_hbm.at[pl.ds(b0, BW), :], bidx_v)

    iota = lax.iota(jnp.int32, L)
    iotas = [iota + L * k for k in range(BW // L)]

    def fire(j, m):
        jv = iota * 0 + j
        for k in range(BW // L):
            v = plsc.load_gather(bidx_v, [iotas[k], jv])
            pidx_v[m, pl.ds(L * k, L)] = lax.shift_right_logical(v, 1)
            half_v[m, pl.ds(L * k, L)] = lax.shift_left(
                lax.bitwise_and(v, 1), 6
            )
        return pltpu.async_copy(tab_hbm.at[pidx_v.at[m]], rows_v.at[m], gsem.at[m])

    def transpose_store(m):
        halves = [half_v[m, pl.ds(L * k, L)] for k in range(BW // L)]

        @plsc.parallel_loop(0, L, unroll=2)
        def diag_step(s):
            diag = lax.bitwise_and(iota + s, L - 1)
            for t in range(D // L):
                dvec = diag + (L * t)
                for k in range(BW // L):
                    val = plsc.load_gather(
                        rows_v.at[m], [iotas[k], halves[k] + dvec]
                    )
                    plsc.store_scatter(st_v.at[m], [dvec, iotas[k]], val)

    def wdrain(m):
        pltpu.make_async_copy(
            st_v.at[m], out_hbm.at[0, :, pl.ds(b0, BW)], wsem.at[m]
        ).wait()

    def group(g, _):
        j0 = g * NBUF
        handles = [fire(j0 + m, m) for m in range(NBUF)]
        for m in range(NBUF):
            handles[m].wait()

            @pl.when(g > 0)
            def _():
                wdrain(m)

            transpose_store(m)
            pltpu.async_copy(
                st_v.at[m], out_hbm.at[j0 + m, :, pl.ds(b0, BW)], wsem.at[m]
            )
        return 0

    lax.fori_loop(0, S // NBUF, group, 0)
    for m in range(NBUF):
        wdrain(m)


@jax.jit
def kernel(token_ids, table):
    tab2 = table.astype(jnp.float32).reshape(50000, 128)
    run = functools.partial(
        pl.kernel,
        out_type=jax.ShapeDtypeStruct((S, D, NB), jnp.float32),
        mesh=plsc.VectorSubcoreMesh(core_axis_name="c", subcore_axis_name="s"),
        scratch_types=[
            pltpu.VMEM((BW, S), jnp.int32),
            pltpu.VMEM((NBUF, BW), jnp.int32),
            pltpu.VMEM((NBUF, BW), jnp.int32),
            pltpu.VMEM((NBUF, BW, 128), jnp.float32),
            pltpu.VMEM((NBUF, D, BW), jnp.float32),
            pltpu.SemaphoreType.DMA((NBUF,)),
            pltpu.SemaphoreType.DMA((NBUF,)),
        ],
        compiler_params=pltpu.CompilerParams(
            use_tc_tiling_on_sc=True, needs_layout_passes=False
        ),
    )(_emb_body)
    out = run(token_ids.astype(jnp.int32), tab2)
    return jnp.transpose(out, (2, 0, 1))


# speedup vs baseline: 1.1555x; 1.1555x over previous
import functools

import jax
import jax.numpy as jnp
from jax import lax
from jax.experimental import pallas as pl
from jax.experimental.pallas import tpu as pltpu
from jax.experimental.pallas import tpu_sc as plsc

NC = 2
NS = 16
NW = NC * NS

D = 64
NB = 4096
S = 50
BW = NB // NW
NBUF = 2
L = 16


def _emb_body(idx_hbm, tab_hbm, out_hbm, idx_v, pidx_v, rows_v, st_v, gsem, wsem):
    wid = lax.axis_index("s") * NC + lax.axis_index("c")
    b0 = wid * BW
    pltpu.sync_copy(idx_hbm.at[:, pl.ds(b0, BW)], idx_v)

    iotas = [lax.iota(jnp.int32, L) + L * k for k in range(BW // L)]

    def fire(j, m):
        for k in range(BW // L):
            v = idx_v[j, pl.ds(L * k, L)]
            pidx_v[m, pl.ds(L * k, L)] = lax.shift_right_logical(v, 1)
        pltpu.async_copy(tab_hbm.at[pidx_v.at[m]], rows_v.at[m], gsem.at[m])

    def gwait(m):
        pltpu.make_async_copy(
            tab_hbm.at[pidx_v.at[m]], rows_v.at[m], gsem.at[m]
        ).wait()

    def transpose_store(j, m):
        halves = [
            lax.shift_left(
                lax.bitwise_and(idx_v[j, pl.ds(L * k, L)], 1), 6
            )
            for k in range(BW // L)
        ]

        iota = lax.iota(jnp.int32, L)

        @plsc.parallel_loop(0, L, unroll=2)
        def diag_step(s):
            diag = lax.bitwise_and(iota + s, L - 1)
            for t in range(D // L):
                dvec = diag + (L * t)
                for k in range(BW // L):
                    val = plsc.load_gather(
                        rows_v.at[m], [iotas[k], halves[k] + dvec]
                    )
                    plsc.store_scatter(
                        st_v.at[m], [dvec, iotas[k]], val
                    )

    def wdrain(m):
        pltpu.make_async_copy(
            st_v.at[m], out_hbm.at[0, :, pl.ds(b0, BW)], wsem.at[m]
        ).wait()

    for m in range(NBUF):
        fire(m, m)

    def group(g, _):
        j0 = g * NBUF
        for m in range(NBUF):
            gwait(m)

            @pl.when(g > 0)
            def _():
                wdrain(m)

            transpose_store(j0 + m, m)

            @pl.when(g + 1 < S // NBUF)
            def _():
                fire(j0 + NBUF + m, m)

            pltpu.async_copy(
                st_v.at[m], out_hbm.at[j0 + m, :, pl.ds(b0, BW)], wsem.at[m]
            )
        return 0

    lax.fori_loop(0, S // NBUF, group, 0)
    for m in range(NBUF):
        wdrain(m)


@jax.jit
def kernel(token_ids, table):
    idx_t = token_ids.astype(jnp.int32).T
    tab2 = table.astype(jnp.float32).reshape(50000, 128)
    run = functools.partial(
        pl.kernel,
        out_type=jax.ShapeDtypeStruct((S, D, NB), jnp.float32),
        mesh=plsc.VectorSubcoreMesh(core_axis_name="c", subcore_axis_name="s"),
        scratch_types=[
            pltpu.VMEM((S, BW), jnp.int32),
            pltpu.VMEM((NBUF, BW), jnp.int32),
            pltpu.VMEM((NBUF, BW, 128), jnp.float32),
            pltpu.VMEM((NBUF, D, BW), jnp.float32),
            pltpu.SemaphoreType.DMA((NBUF,)),
            pltpu.SemaphoreType.DMA((NBUF,)),
        ],
        compiler_params=pltpu.CompilerParams(
            use_tc_tiling_on_sc=True, needs_layout_passes=False
        ),
    )(_emb_body)
    out = run(idx_t, tab2)
    return jnp.transpose(out, (2, 0, 1))


# speedup vs baseline: 1.1834x; 1.0242x over previous
import functools

import jax
import jax.numpy as jnp
from jax import lax
from jax.experimental import pallas as pl
from jax.experimental.pallas import tpu as pltpu
from jax.experimental.pallas import tpu_sc as plsc

NC = 2
NS = 16
NW = NC * NS

D = 64
NB = 4096
S = 50
BW = NB // NW
NBUF = 3
L = 16


def _emb_body(idx_hbm, tab_hbm, out_hbm, idx_v, pidx_v, rows_v, st_v, gsem, wsem):
    wid = lax.axis_index("s") * NC + lax.axis_index("c")
    b0 = wid * BW
    pltpu.sync_copy(idx_hbm.at[:, pl.ds(b0, BW)], idx_v)

    iotas = [lax.iota(jnp.int32, L) + L * k for k in range(BW // L)]

    def fire(j, m):
        for k in range(BW // L):
            v = idx_v[j, pl.ds(L * k, L)]
            pidx_v[m, pl.ds(L * k, L)] = lax.shift_right_logical(v, 1)
        pltpu.async_copy(tab_hbm.at[pidx_v.at[m]], rows_v.at[m], gsem.at[m])

    def gwait(m):
        pltpu.make_async_copy(
            tab_hbm.at[pidx_v.at[m]], rows_v.at[m], gsem.at[m]
        ).wait()

    def transpose_store(j, m):
        halves = [
            lax.shift_left(
                lax.bitwise_and(idx_v[j, pl.ds(L * k, L)], 1), 6
            )
            for k in range(BW // L)
        ]

        iota = lax.iota(jnp.int32, L)

        @plsc.parallel_loop(0, L, unroll=2)
        def diag_step(s):
            diag = lax.bitwise_and(iota + s, L - 1)
            for t in range(D // L):
                dvec = diag + (L * t)
                for k in range(BW // L):
                    val = plsc.load_gather(
                        rows_v.at[m], [iotas[k], halves[k] + dvec]
                    )
                    plsc.store_scatter(
                        st_v.at[m], [dvec, iotas[k]], val
                    )

    def wdrain(m):
        pltpu.make_async_copy(
            st_v.at[m], out_hbm.at[0, :, pl.ds(b0, BW)], wsem.at[m]
        ).wait()

    for m in range(NBUF):
        fire(m, m)

    def group(g, _):
        j0 = g * NBUF
        for m in range(NBUF):
            gwait(m)

            @pl.when(g > 0)
            def _():
                wdrain(m)

            transpose_store(j0 + m, m)

            @pl.when(j0 + NBUF + m < S)
            def _():
                fire(j0 + NBUF + m, m)

            pltpu.async_copy(
                st_v.at[m], out_hbm.at[j0 + m, :, pl.ds(b0, BW)], wsem.at[m]
            )
        return 0

    lax.fori_loop(0, S // NBUF, group, 0)
    for m in range(S % NBUF):
        j = S - S % NBUF + m
        gwait(m)
        wdrain(m)
        transpose_store(j, m)
        pltpu.async_copy(
            st_v.at[m], out_hbm.at[j, :, pl.ds(b0, BW)], wsem.at[m]
        )
    for m in range(NBUF):
        wdrain(m)


@jax.jit
def kernel(token_ids, table):
    idx_t = token_ids.astype(jnp.int32).T
    tab2 = table.astype(jnp.float32).reshape(50000, 128)
    run = functools.partial(
        pl.kernel,
        out_type=jax.ShapeDtypeStruct((S, D, NB), jnp.float32),
        mesh=plsc.VectorSubcoreMesh(core_axis_name="c", subcore_axis_name="s"),
        scratch_types=[
            pltpu.VMEM((S, BW), jnp.int32),
            pltpu.VMEM((NBUF, BW), jnp.int32),
            pltpu.VMEM((NBUF, BW, 128), jnp.float32),
            pltpu.VMEM((NBUF, D, BW), jnp.float32),
            pltpu.SemaphoreType.DMA((NBUF,)),
            pltpu.SemaphoreType.DMA((NBUF,)),
        ],
        compiler_params=pltpu.CompilerParams(
            use_tc_tiling_on_sc=True, needs_layout_passes=False
        ),
    )(_emb_body)
    out = run(idx_t, tab2)
    return jnp.transpose(out, (2, 0, 1))
